# baseline (speedup 1.0000x reference)
import functools

import jax
import jax.numpy as jnp
from jax import lax
from jax.experimental import pallas as pl
from jax.experimental.pallas import tpu as pltpu
from jax.experimental.pallas import tpu_sc as plsc

N = 10000
E = 320000
D = 128

NC = 2
NS = 16
L = 16
NW = NC * NS
EPW = E // NW
CH = 80
CPW = EPW // CH
NB = 4
RPB = 624
TAIL = N - NS * RPB

BLK = 1000


def _edge_pass(d_feat):
    mesh = plsc.VectorSubcoreMesh(core_axis_name="c", subcore_axis_name="s")

    @functools.partial(
        pl.kernel,
        mesh=mesh,
        out_type=jax.ShapeDtypeStruct((NC, N, d_feat), jnp.float32),
        scratch_types=[
            pltpu.VMEM((NB, CH), jnp.int32),
            pltpu.VMEM((NB, CH), jnp.int32),
            pltpu.VMEM((NB, CH, d_feat), jnp.float32),
            pltpu.VMEM_SHARED((N, d_feat), jnp.float32),
            pltpu.SemaphoreType.DMA((NB,)),
            pltpu.SemaphoreType.DMA((NB,)),
            pltpu.SemaphoreType.DMA((NB,)),
            pltpu.SemaphoreType.DMA,
        ],
    )
    def edge_pass(s_hbm, src_hbm, dst_hbm, z_hbm, zz_hbm, out_hbm,
                  srcs_r, dsts_r, rows_v, acc_sh, isem, gsem, ssem, zsem):
        cid = lax.axis_index("c")
        sid = lax.axis_index("s")
        wid = sid * NC + cid

        @pl.when(cid == 0)
        def _():
            pltpu.async_copy(z_hbm.at[pl.ds(sid * RPB, RPB)],
                             acc_sh.at[pl.ds(sid * RPB, RPB)], zsem)

            @pl.when(sid == NS - 1)
            def _():
                pltpu.async_copy(z_hbm.at[pl.ds(NS * RPB, TAIL)],
                                 acc_sh.at[pl.ds(NS * RPB, TAIL)], zsem)

        @pl.when(cid == 1)
        def _():
            pltpu.async_copy(zz_hbm.at[pl.ds(0, RPB)],
                             acc_sh.at[pl.ds(sid * RPB, RPB)], zsem)

            @pl.when(sid == NS - 1)
            def _():
                pltpu.async_copy(zz_hbm.at[pl.ds(0, TAIL)],
                                 acc_sh.at[pl.ds(NS * RPB, TAIL)], zsem)

        ebase = wid * EPW

        def body(i, carry):
            @pl.when(i < CPW)
            def _():
                b = i % NB

                @pl.when(i >= NB)
                def _():
                    pltpu.make_async_copy(
                        rows_v.at[b], acc_sh.at[dsts_r.at[b]],
                        ssem.at[b]).wait()

                off = ebase + i * CH
                pltpu.async_copy(src_hbm.at[pl.ds(off, CH)], srcs_r.at[b],
                                 isem.at[b])
                pltpu.async_copy(dst_hbm.at[pl.ds(off, CH)], dsts_r.at[b],
                                 isem.at[b])

            j = i - 1

            @pl.when(jnp.logical_and(j >= 0, j < CPW))
            def _():
                bj = j % NB
                offj = ebase + j * CH
                pltpu.make_async_copy(src_hbm.at[pl.ds(offj, CH)],
                                      srcs_r.at[bj], isem.at[bj]).wait()
                pltpu.make_async_copy(dst_hbm.at[pl.ds(offj, CH)],
                                      dsts_r.at[bj], isem.at[bj]).wait()
                pltpu.async_copy(s_hbm.at[srcs_r.at[bj]], rows_v.at[bj],
                                 gsem.at[bj])

            k = i - 2

            @pl.when(k == 0)
            def _():
                pltpu.make_async_copy(
                    zz_hbm.at[pl.ds(0, RPB)],
                    acc_sh.at[pl.ds(sid * RPB, RPB)], zsem).wait()

                @pl.when(sid == NS - 1)
                def _():
                    pltpu.make_async_copy(
                        zz_hbm.at[pl.ds(0, TAIL)],
                        acc_sh.at[pl.ds(NS * RPB, TAIL)], zsem).wait()

                plsc.subcore_barrier()

            @pl.when(jnp.logical_and(k >= 0, k < CPW))
            def _():
                bk = k % NB
                pltpu.make_async_copy(s_hbm.at[srcs_r.at[bk]],
                                      rows_v.at[bk], gsem.at[bk]).wait()
                pltpu.async_copy(rows_v.at[bk], acc_sh.at[dsts_r.at[bk]],
                                 ssem.at[bk], add=True)

            return carry

        lax.fori_loop(0, CPW + 2, body, 0)

        def drain(t, carry):
            b = (CPW - NB + t) % NB
            pltpu.make_async_copy(rows_v.at[b], acc_sh.at[dsts_r.at[b]],
                                  ssem.at[b]).wait()
            return carry

        lax.fori_loop(0, NB, drain, 0)
        plsc.subcore_barrier()

        pltpu.sync_copy(acc_sh.at[pl.ds(sid * RPB, RPB)],
                        out_hbm.at[cid, pl.ds(sid * RPB, RPB)])

        @pl.when(sid == NS - 1)
        def _():
            pltpu.sync_copy(acc_sh.at[pl.ds(NS * RPB, TAIL)],
                            out_hbm.at[cid, pl.ds(NS * RPB, TAIL)])

    return edge_pass


_edge_pass_128 = _edge_pass(D)


def _tc_support(x, w):
    dn = w.shape[1]

    def body(x_ref, w_ref, s_ref):
        s_ref[...] = jnp.dot(x_ref[...], w_ref[...],
                             preferred_element_type=jnp.float32)

    return pl.pallas_call(
        body,
        grid=(N // BLK,),
        in_specs=[pl.BlockSpec((BLK, D), lambda i: (i, 0)),
                  pl.BlockSpec((D, dn), lambda i: (0, 0))],
        out_specs=pl.BlockSpec((BLK, dn), lambda i: (i, 0)),
        out_shape=jax.ShapeDtypeStruct((N, dn), jnp.float32),
    )(x, w)


def _tc_affine(x, wl, b):
    dn = wl.shape[1]

    def body(x_ref, wl_ref, b_ref, z_ref):
        z_ref[...] = (jnp.dot(x_ref[...], wl_ref[...],
                              preferred_element_type=jnp.float32)
                      + b_ref[...])

    return pl.pallas_call(
        body,
        grid=(N // BLK,),
        in_specs=[pl.BlockSpec((BLK, D), lambda i: (i, 0)),
                  pl.BlockSpec((D, dn), lambda i: (0, 0)),
                  pl.BlockSpec((1, dn), lambda i: (0, 0))],
        out_specs=pl.BlockSpec((BLK, dn), lambda i: (i, 0)),
        out_shape=jax.ShapeDtypeStruct((N, dn), jnp.float32),
    )(x, wl, b.reshape(1, dn))


def _tc_combine(p, wn, res=None):
    dn = wn.shape[1]

    def body(*refs):
        if res is None:
            p_ref, wn_ref, h_ref, s_ref = refs
        else:
            p_ref, res_ref, wn_ref, h_ref, s_ref = refs
        h = jnp.maximum(p_ref[0] + p_ref[1], 0.0)
        if res is not None:
            h = (res_ref[...] + h) * 0.5
        h_ref[...] = h
        s_ref[...] = jnp.dot(h, wn_ref[...],
                             preferred_element_type=jnp.float32)

    in_specs = [pl.BlockSpec((NC, BLK, D), lambda i: (0, i, 0))]
    args = [p]
    if res is not None:
        in_specs.append(pl.BlockSpec((BLK, D), lambda i: (i, 0)))
        args.append(res)
    in_specs.append(pl.BlockSpec((D, dn), lambda i: (0, 0)))
    args.append(wn)

    return pl.pallas_call(
        body,
        grid=(N // BLK,),
        in_specs=in_specs,
        out_specs=[pl.BlockSpec((BLK, D), lambda i: (i, 0)),
                   pl.BlockSpec((BLK, dn), lambda i: (i, 0))],
        out_shape=[jax.ShapeDtypeStruct((N, D), jnp.float32),
                   jax.ShapeDtypeStruct((N, dn), jnp.float32)],
    )(*args)


def _tc_final(p):
    def body(p_ref, o_ref):
        o_ref[...] = p_ref[0] + p_ref[1]

    return pl.pallas_call(
        body,
        grid=(N // BLK,),
        in_specs=[pl.BlockSpec((NC, BLK, D), lambda i: (0, i, 0))],
        out_specs=pl.BlockSpec((BLK, D), lambda i: (i, 0)),
        out_shape=jax.ShapeDtypeStruct((N, D), jnp.float32),
    )(p)


def kernel(inputs, edge_index, W1, W1l, b1, Wb1, Wb1l, bb1,
           Wb2, Wb2l, bb2, W2, W2l, b2):
    x = inputs
    src = edge_index[0]
    dst = edge_index[1]

    zz = jnp.zeros((RPB, D), jnp.float32)

    w2p = jnp.pad(W2, ((0, 0), (0, D - W2.shape[1])))
    w2lp = jnp.pad(W2l, ((0, 0), (0, D - W2l.shape[1])))
    b2p = jnp.pad(b2, (0, D - b2.shape[0]))

    s = _tc_support(x, W1)
    z = _tc_affine(x, W1l, b1)
    p = _edge_pass_128(s, src, dst, z, zz)
    h, s = _tc_combine(p, Wb1[0])

    for i in range(3):
        z = _tc_affine(h, Wb1l[i], bb1[i])
        p = _edge_pass_128(s, src, dst, z, zz)
        t, s = _tc_combine(p, Wb2[i])
        z = _tc_affine(t, Wb2l[i], bb2[i])
        p = _edge_pass_128(s, src, dst, z, zz)
        wn = Wb1[i + 1] if i < 2 else w2p
        h, s = _tc_combine(p, wn, res=h)

    z = _tc_affine(h, w2lp, b2p)
    p = _edge_pass_128(s, src, dst, z, zz)
    out_pad = _tc_final(p)
    return (out_pad[:, :3], h)


# speedup vs baseline: 1.0860x; 1.0860x over previous
import functools

import jax
import jax.numpy as jnp
from jax import lax
from jax.experimental import pallas as pl
from jax.experimental.pallas import tpu as pltpu
from jax.experimental.pallas import tpu_sc as plsc

N = 10000
E = 320000
D = 128

NC = 2
NS = 16
L = 16
NW = NC * NS
EPW = E // NW
CH = 80
CPW = EPW // CH
NB = 4
RPB = 624
TAIL = N - NS * RPB
ZR = 48

BLK = 1000


def _edge_pass(d_feat):
    mesh = plsc.VectorSubcoreMesh(core_axis_name="c", subcore_axis_name="s")

    @functools.partial(
        pl.kernel,
        mesh=mesh,
        out_type=jax.ShapeDtypeStruct((NC, N, d_feat), jnp.float32),
        scratch_types=[
            pltpu.VMEM((NB, CH), jnp.int32),
            pltpu.VMEM((NB, CH), jnp.int32),
            pltpu.VMEM((NB, CH, d_feat), jnp.float32),
            pltpu.VMEM((ZR, d_feat), jnp.float32),
            pltpu.VMEM_SHARED((N, d_feat), jnp.float32),
            pltpu.SemaphoreType.DMA((NB,)),
            pltpu.SemaphoreType.DMA((NB,)),
            pltpu.SemaphoreType.DMA((NB,)),
            pltpu.SemaphoreType.DMA,
        ],
    )
    def edge_pass(s_hbm, src_hbm, dst_hbm, out_hbm,
                  srcs_r, dsts_r, rows_v, zbuf_v, acc_sh,
                  isem, gsem, ssem, zsem):
        cid = lax.axis_index("c")
        sid = lax.axis_index("s")
        wid = sid * NC + cid

        zero = jnp.zeros((L,), jnp.float32)
        sub = d_feat // L

        def zfill(i, carry):
            zbuf_v[i // sub, pl.ds((i % sub) * L, L)] = zero
            return carry

        lax.fori_loop(0, ZR * sub, zfill, 0)

        def zslice(j, carry):
            pltpu.async_copy(zbuf_v,
                             acc_sh.at[pl.ds(sid * RPB + j * ZR, ZR)], zsem)
            return carry

        lax.fori_loop(0, RPB // ZR, zslice, 0)

        @pl.when(sid == NS - 1)
        def _():
            pltpu.async_copy(zbuf_v.at[pl.ds(0, TAIL)],
                             acc_sh.at[pl.ds(NS * RPB, TAIL)], zsem)

        ebase = wid * EPW

        def body(i, carry):
            @pl.when(i < CPW)
            def _():
                b = i % NB

                @pl.when(i >= NB)
                def _():
                    pltpu.make_async_copy(
                        rows_v.at[b], acc_sh.at[dsts_r.at[b]],
                        ssem.at[b]).wait()

                off = ebase + i * CH
                pltpu.async_copy(src_hbm.at[pl.ds(off, CH)], srcs_r.at[b],
                                 isem.at[b])
                pltpu.async_copy(dst_hbm.at[pl.ds(off, CH)], dsts_r.at[b],
                                 isem.at[b])

            j = i - 1

            @pl.when(jnp.logical_and(j >= 0, j < CPW))
            def _():
                bj = j % NB
                offj = ebase + j * CH
                pltpu.make_async_copy(src_hbm.at[pl.ds(offj, CH)],
                                      srcs_r.at[bj], isem.at[bj]).wait()
                pltpu.make_async_copy(dst_hbm.at[pl.ds(offj, CH)],
                                      dsts_r.at[bj], isem.at[bj]).wait()
                pltpu.async_copy(s_hbm.at[srcs_r.at[bj]], rows_v.at[bj],
                                 gsem.at[bj])

            k = i - 2

            @pl.when(k == 0)
            def _():
                def zwait(j2, carry2):
                    pltpu.make_async_copy(
                        zbuf_v,
                        acc_sh.at[pl.ds(sid * RPB + j2 * ZR, ZR)],
                        zsem).wait()
                    return carry2

                lax.fori_loop(0, RPB // ZR, zwait, 0)

                @pl.when(sid == NS - 1)
                def _():
                    pltpu.make_async_copy(
                        zbuf_v.at[pl.ds(0, TAIL)],
                        acc_sh.at[pl.ds(NS * RPB, TAIL)], zsem).wait()

                plsc.subcore_barrier()

            @pl.when(jnp.logical_and(k >= 0, k < CPW))
            def _():
                bk = k % NB
                pltpu.make_async_copy(s_hbm.at[srcs_r.at[bk]],
                                      rows_v.at[bk], gsem.at[bk]).wait()
                pltpu.async_copy(rows_v.at[bk], acc_sh.at[dsts_r.at[bk]],
                                 ssem.at[bk], add=True)

            return carry

        lax.fori_loop(0, CPW + 2, body, 0)

        def drain(t, carry):
            b = (CPW - NB + t) % NB
            pltpu.make_async_copy(rows_v.at[b], acc_sh.at[dsts_r.at[b]],
                                  ssem.at[b]).wait()
            return carry

        lax.fori_loop(0, NB, drain, 0)
        plsc.subcore_barrier()

        pltpu.sync_copy(acc_sh.at[pl.ds(sid * RPB, RPB)],
                        out_hbm.at[cid, pl.ds(sid * RPB, RPB)])

        @pl.when(sid == NS - 1)
        def _():
            pltpu.sync_copy(acc_sh.at[pl.ds(NS * RPB, TAIL)],
                            out_hbm.at[cid, pl.ds(NS * RPB, TAIL)])

    return edge_pass


_edge_pass_128 = _edge_pass(D)


def _tc_support(x, w):
    dn = w.shape[1]

    def body(x_ref, w_ref, s_ref):
        s_ref[...] = jnp.dot(x_ref[...], w_ref[...],
                             preferred_element_type=jnp.float32)

    return pl.pallas_call(
        body,
        grid=(N // BLK,),
        in_specs=[pl.BlockSpec((BLK, D), lambda i: (i, 0)),
                  pl.BlockSpec((D, dn), lambda i: (0, 0))],
        out_specs=pl.BlockSpec((BLK, dn), lambda i: (i, 0)),
        out_shape=jax.ShapeDtypeStruct((N, dn), jnp.float32),
    )(x, w)


def _tc_affine(x, wl, b):
    dn = wl.shape[1]

    def body(x_ref, wl_ref, b_ref, z_ref):
        z_ref[...] = (jnp.dot(x_ref[...], wl_ref[...],
                              preferred_element_type=jnp.float32)
                      + b_ref[...])

    return pl.pallas_call(
        body,
        grid=(N // BLK,),
        in_specs=[pl.BlockSpec((BLK, D), lambda i: (i, 0)),
                  pl.BlockSpec((D, dn), lambda i: (0, 0)),
                  pl.BlockSpec((1, dn), lambda i: (0, 0))],
        out_specs=pl.BlockSpec((BLK, dn), lambda i: (i, 0)),
        out_shape=jax.ShapeDtypeStruct((N, dn), jnp.float32),
    )(x, wl, b.reshape(1, dn))


def _tc_combine(p, z, wn, res=None):
    dn = wn.shape[1]

    def body(*refs):
        if res is None:
            p_ref, z_ref, wn_ref, h_ref, s_ref = refs
        else:
            p_ref, z_ref, res_ref, wn_ref, h_ref, s_ref = refs
        h = jnp.maximum(p_ref[0] + p_ref[1] + z_ref[...], 0.0)
        if res is not None:
            h = (res_ref[...] + h) * 0.5
        h_ref[...] = h
        s_ref[...] = jnp.dot(h, wn_ref[...],
                             preferred_element_type=jnp.float32)

    in_specs = [pl.BlockSpec((NC, BLK, D), lambda i: (0, i, 0)),
                pl.BlockSpec((BLK, D), lambda i: (i, 0))]
    args = [p, z]
    if res is not None:
        in_specs.append(pl.BlockSpec((BLK, D), lambda i: (i, 0)))
        args.append(res)
    in_specs.append(pl.BlockSpec((D, dn), lambda i: (0, 0)))
    args.append(wn)

    return pl.pallas_call(
        body,
        grid=(N // BLK,),
        in_specs=in_specs,
        out_specs=[pl.BlockSpec((BLK, D), lambda i: (i, 0)),
                   pl.BlockSpec((BLK, dn), lambda i: (i, 0))],
        out_shape=[jax.ShapeDtypeStruct((N, D), jnp.float32),
                   jax.ShapeDtypeStruct((N, dn), jnp.float32)],
    )(*args)


def _tc_final(p, z):
    def body(p_ref, z_ref, o_ref):
        o_ref[...] = p_ref[0] + p_ref[1] + z_ref[...]

    return pl.pallas_call(
        body,
        grid=(N // BLK,),
        in_specs=[pl.BlockSpec((NC, BLK, D), lambda i: (0, i, 0)),
                  pl.BlockSpec((BLK, D), lambda i: (i, 0))],
        out_specs=pl.BlockSpec((BLK, D), lambda i: (i, 0)),
        out_shape=jax.ShapeDtypeStruct((N, D), jnp.float32),
    )(p, z)


def kernel(inputs, edge_index, W1, W1l, b1, Wb1, Wb1l, bb1,
           Wb2, Wb2l, bb2, W2, W2l, b2):
    x = inputs
    src = edge_index[0]
    dst = edge_index[1]

    w2p = jnp.pad(W2, ((0, 0), (0, D - W2.shape[1])))
    w2lp = jnp.pad(W2l, ((0, 0), (0, D - W2l.shape[1])))
    b2p = jnp.pad(b2, (0, D - b2.shape[0]))

    s = _tc_support(x, W1)
    z = _tc_affine(x, W1l, b1)
    p = _edge_pass_128(s, src, dst)
    h, s = _tc_combine(p, z, Wb1[0])

    for i in range(3):
        z = _tc_affine(h, Wb1l[i], bb1[i])
        p = _edge_pass_128(s, src, dst)
        t, s = _tc_combine(p, z, Wb2[i])
        z = _tc_affine(t, Wb2l[i], bb2[i])
        p = _edge_pass_128(s, src, dst)
        wn = Wb1[i + 1] if i < 2 else w2p
        h, s = _tc_combine(p, z, wn, res=h)

    z = _tc_affine(h, w2lp, b2p)
    p = _edge_pass_128(s, src, dst)
    out_pad = _tc_final(p, z)
    return (out_pad[:, :3], h)
